# baseline (speedup 1.0000x reference)
import functools

import jax
import jax.numpy as jnp
from jax import lax
from jax.experimental import pallas as pl
from jax.experimental.pallas import tpu as pltpu
from jax.experimental.pallas import tpu_sc as plsc

NC = 2
NS = 16
NW = NC * NS
CH = 64
LANES = 16
NBUF = 5


def _mm_body(x_ref, t_ref, o0_ref, o1_ref, o2_ref):
    y = lax.dot_general(x_ref[...], t_ref[...], (((0,), (0,)), ((), ())),
                        preferred_element_type=jnp.float32)
    c = o0_ref.shape[1]
    o0_ref[...] = y[:, 0:c]
    o1_ref[...] = y[:, c:2 * c]
    o2_ref[...] = y[:, 2 * c:3 * c]


def _mm3(x2d, thetaT):
    cin, m = x2d.shape
    ck3 = thetaT.shape[1]
    c = ck3 // 3
    out = jax.ShapeDtypeStruct((m, c), jnp.float32)
    return pl.pallas_call(
        _mm_body,
        grid=(1,),
        in_specs=[
            pl.BlockSpec((cin, m), lambda i: (0, 0)),
            pl.BlockSpec((cin, ck3), lambda i: (0, 0)),
        ],
        out_specs=[pl.BlockSpec((m, c), lambda i: (0, 0))] * 3,
        out_shape=[out, out, out],
    )(x2d, thetaT)


def _add3_body(a_ref, b_ref, c_ref, d_ref, o_ref):
    o_ref[...] = a_ref[...] + b_ref[...] + c_ref[...] + d_ref[...]


def _add3(a, b, c, brow, bm=2000):
    m, ch = a.shape
    spec = pl.BlockSpec((bm, ch), lambda i: (i, 0))
    return pl.pallas_call(
        _add3_body,
        grid=(m // bm,),
        in_specs=[spec, spec, spec, pl.BlockSpec((1, ch), lambda i: (0, 0))],
        out_specs=spec,
        out_shape=jax.ShapeDtypeStruct((m, ch), jnp.float32),
    )(a, b, c, brow)


def _vgather(vec, idx16):
    dnums = lax.GatherDimensionNumbers(
        offset_dims=(), collapsed_slice_dims=(0,), start_index_map=(0,))
    return lax.gather(vec, idx16[:, None], dnums, (1,),
                      mode=lax.GatherScatterMode.PROMISE_IN_BOUNDS)


def _spmm_partials(rows, cols, vals, table, zinit):
    nnzp = vals.shape[0]
    m, c = table.shape
    nt = nnzp // (NW * CH)
    assert nt % NBUF == 0
    rpt = 8 * (m // 8 // NS)
    rem = m - NS * rpt

    mesh = plsc.VectorSubcoreMesh(core_axis_name="c", subcore_axis_name="s")

    scratch = (
        [pltpu.VMEM((CH, c), jnp.float32) for _ in range(NBUF)]
        + [pltpu.VMEM((CH,), jnp.int32) for _ in range(NBUF)]
        + [pltpu.VMEM((CH,), jnp.int32) for _ in range(NBUF)]
        + [pltpu.VMEM((CH,), jnp.float32) for _ in range(NBUF)]
        + [pltpu.VMEM_SHARED((m, c), jnp.float32)]
        + [pltpu.SemaphoreType.DMA] * (3 * NBUF)
    )

    @functools.partial(
        pl.kernel,
        out_type=jax.ShapeDtypeStruct((NC, m, c), jnp.float32),
        mesh=mesh,
        scratch_types=scratch,
    )
    def spmm(rows_hbm, cols_hbm, vals_hbm, table_hbm, zinit_hbm, out_hbm, *sc):
        gath = sc[0:NBUF]
        rowv = sc[NBUF:2 * NBUF]
        colv = sc[2 * NBUF:3 * NBUF]
        valv = sc[3 * NBUF:4 * NBUF]
        acc = sc[4 * NBUF]
        semi = sc[4 * NBUF + 1:4 * NBUF + 1 + NBUF]
        semg = sc[4 * NBUF + 1 + NBUF:4 * NBUF + 1 + 2 * NBUF]
        sems = sc[4 * NBUF + 1 + 2 * NBUF:4 * NBUF + 1 + 3 * NBUF]

        cid = lax.axis_index("c")
        sid = lax.axis_index("s")
        wid = cid * NS + sid
        base = wid * nt * CH

        pltpu.sync_copy(zinit_hbm.at[pl.ds(0, rpt)],
                        acc.at[pl.ds(sid * rpt, rpt)])
        if rem:
            @pl.when(sid == NS - 1)
            def _():
                pltpu.sync_copy(zinit_hbm.at[pl.ds(0, rem)],
                                acc.at[pl.ds(NS * rpt, rem)])
        plsc.subcore_barrier()

        def fire_idx(t, b):
            e0 = base + t * CH
            pltpu.async_copy(rows_hbm.at[pl.ds(e0, CH)], rowv[b], semi[b])
            pltpu.async_copy(cols_hbm.at[pl.ds(e0, CH)], colv[b], semi[b])
            pltpu.async_copy(vals_hbm.at[pl.ds(e0, CH)], valv[b], semi[b])

        def wait_idx(b):
            pltpu.make_async_copy(rows_hbm.at[pl.ds(0, CH)], rowv[b],
                                  semi[b]).wait()
            pltpu.make_async_copy(cols_hbm.at[pl.ds(0, CH)], colv[b],
                                  semi[b]).wait()
            pltpu.make_async_copy(vals_hbm.at[pl.ds(0, CH)], valv[b],
                                  semi[b]).wait()

        def fire_gather(b):
            pltpu.async_copy(table_hbm.at[colv[b]], gath[b], semg[b])

        def wait_gather(b):
            pltpu.make_async_copy(table_hbm.at[colv[b]], gath[b],
                                  semg[b]).wait()

        def fire_scatter(b):
            pltpu.async_copy(gath[b], acc.at[rowv[b]], sems[b], add=True)

        def wait_scatter(b):
            pltpu.make_async_copy(gath[b], acc.at[rowv[b]], sems[b]).wait()

        def scale(b):
            def scale_block(eb, cc):
                vblock = valv[b][pl.ds(eb * LANES, LANES)]
                for l in range(LANES):
                    vv = _vgather(vblock, jnp.full((LANES,), l, jnp.int32))
                    e = eb * LANES + l
                    for j in range(c // LANES):
                        g = gath[b][e, pl.ds(j * LANES, LANES)]
                        gath[b][e, pl.ds(j * LANES, LANES)] = g * vv
                return cc

            lax.fori_loop(0, CH // LANES, scale_block, 0)

        fire_idx(0, 0)
        fire_idx(1, 1)
        wait_idx(0)
        fire_gather(0)

        def group(outer, carry):
            s0 = outer * NBUF
            for g in range(NBUF):
                s = s0 + g
                ba = (g + 2) % NBUF

                @pl.when(s + 2 < nt)
                def _():
                    @pl.when(s >= 3)
                    def _():
                        wait_scatter(ba)
                    fire_idx(s + 2, ba)

                bb = (g + 1) % NBUF

                @pl.when(s + 1 < nt)
                def _():
                    wait_idx(bb)
                    fire_gather(bb)

                wait_gather(g)
                scale(g)
                fire_scatter(g)
            return carry

        lax.fori_loop(0, nt // NBUF, group, 0)

        for b in range(NBUF):
            wait_scatter(b)

        plsc.subcore_barrier()

        pltpu.sync_copy(acc.at[pl.ds(sid * rpt, rpt)],
                        out_hbm.at[cid, pl.ds(sid * rpt, rpt)])
        if rem:
            @pl.when(sid == NS - 1)
            def _():
                pltpu.sync_copy(acc.at[pl.ds(NS * rpt, rem)],
                                out_hbm.at[cid, pl.ds(NS * rpt, rem)])

    return spmm(rows, cols, vals, table, zinit)


def kernel(L_indices, L_values, x, theta, bias):
    rows = L_indices[0].astype(jnp.int32)
    cols = L_indices[1].astype(jnp.int32)
    vals = L_values.astype(jnp.float32)

    cout, cin, k = theta.shape
    m = x.shape[2]
    nnz = vals.shape[0]

    quant = NW * CH * NBUF
    nnzp = ((nnz + quant - 1) // quant) * quant
    pad = nnzp - nnz
    if pad:
        rows = jnp.concatenate([rows, jnp.zeros((pad,), jnp.int32)])
        cols = jnp.concatenate([cols, jnp.zeros((pad,), jnp.int32)])
        vals = jnp.concatenate([vals, jnp.zeros((pad,), jnp.float32)])

    thetaT = jnp.transpose(theta, (1, 2, 0)).reshape(cin, k * cout)
    z0, z1, z2 = _mm3(x[0], thetaT)

    zinit = jnp.zeros((640, cout), jnp.float32)
    zrow = jnp.zeros((1, cout), jnp.float32)
    biasT = bias[0, :, 0][None, :]

    u_p = _spmm_partials(rows, cols, vals, z2, zinit)
    u = _add3(u_p[0], u_p[1], z1, zrow)
    y_p = _spmm_partials(rows, cols, vals, u, zinit)
    yT = _add3(y_p[0], y_p[1], z0, biasT)
    return yT.T[None]
